# baseline (speedup 1.0000x reference)
import functools

import jax
import jax.numpy as jnp
from jax.experimental import pallas as pl
from jax.experimental.pallas import tpu as pltpu

_B, _N, _D = 64, 256, 256
_NH = 4
_HD = _D // _NH
_THR = 20.0
_KDEF = 4
_KMAX = 6
_NC = 7
_NCP = 8
_HJ = _NH * _NCP
_H1 = _NCP * (_D // 4)


def _select_body(dist_ref, speed_ref, w1b_ref, bd1t_ref,
                 idx_ref, kdvb_ref, k_ref):
    d0 = dist_ref[...]
    close = jnp.sum((d0 < _THR).astype(jnp.float32), axis=1, keepdims=True)
    avg_density = jnp.mean(close) / d0.shape[1]
    avg_speed = jnp.mean(speed_ref[...])
    k = _KDEF + (avg_speed > 15.0).astype(jnp.int32)
    k = jnp.minimum(k, _KMAX)
    k = jnp.minimum(k + (avg_density > 0.5).astype(jnp.int32), _KMAX)
    k = jnp.minimum(k, d0.shape[1] - 1)
    k_ref[...] = jnp.full((1, 1), k, jnp.int32)

    hd4 = _D // 4
    iota_n = jax.lax.broadcasted_iota(jnp.int32, d0.shape, 1)
    work = d0
    for j in range(_NC):
        mval = jnp.min(work, axis=1, keepdims=True)
        cand = jnp.where(work == mval, iota_n, d0.shape[1])
        midx = jnp.min(cand, axis=1, keepdims=True)
        idx_ref[:, j:j + 1] = midx
        sl = slice(j * hd4, (j + 1) * hd4)
        kdvb_ref[:, sl] = mval * w1b_ref[:, sl] + bd1t_ref[:, sl]
        work = jnp.where(iota_n == midx, jnp.inf, work)
    idx_ref[:, _NC:_NCP] = jnp.zeros((_B, _NCP - _NC), jnp.int32)
    sl = slice(_NC * hd4, _NCP * hd4)
    kdvb_ref[:, sl] = jnp.broadcast_to(bd1t_ref[:, sl], (_B, hd4))


def _attn_body(idx_sref, k_sref,
               x_ref, distT_ref, maskT_ref, kdvb_ref,
               wqe_ref, bq_ref, beq_ref,
               wkv_ref, bk_ref, bv_ref,
               w1a_ref, wbig_ref, bd2big_ref,
               lng_ref, lnb_ref, out_ref, cand_ref):
    b = pl.program_id(0)
    x = x_ref[0]
    cdims = (((1,), (1,)), ((), ()))

    qboth = jax.lax.dot_general(x.astype(jnp.bfloat16), wqe_ref[...], cdims,
                                preferred_element_type=jnp.float32)
    qx = qboth[:, :_D] + bq_ref[...]
    qe = qboth[:, _D:] + beq_ref[...]
    lane = jax.lax.broadcasted_iota(jnp.int32, (_N, _B), 1)
    mcol = jnp.sum(jnp.where(lane == b, maskT_ref[...], 0.0),
                   axis=1, keepdims=True)
    q = qx + mcol * (qe - qx)

    for j in range(_NC):
        cand_ref[j:j + 1, :] = x_ref[0, pl.ds(idx_sref[b, j], 1), :]
    cand_ref[_NC:_NCP, :] = jnp.zeros((_NCP - _NC, _D), jnp.float32)
    cand = cand_ref[...]

    kvboth = jax.lax.dot_general(cand.astype(jnp.bfloat16), wkv_ref[...],
                                 cdims, preferred_element_type=jnp.float32)
    kc = kvboth[:, :_D] + bk_ref[...]
    vc = kvboth[:, _D:] + bv_ref[...]

    hol = jax.lax.broadcasted_iota(jnp.int32, (_NCP, _D), 1) // _HD
    kcbig = jnp.concatenate(
        [jnp.where(hol == h, kc, 0.0) for h in range(_NH)],
        axis=0).astype(jnp.bfloat16)
    vcbig = jnp.concatenate(
        [jnp.where(hol == h, vc, 0.0) for h in range(_NH)],
        axis=0).astype(jnp.bfloat16)

    qd = jnp.sum(jnp.where(lane == b, distT_ref[...], 0.0),
                 axis=1, keepdims=True)
    kdvb = kdvb_ref[pl.ds(b, 1), :]
    h_all = jnp.maximum(qd * w1a_ref[...] + kdvb, 0.0)
    bias_all = jax.lax.dot_general(
        h_all.astype(jnp.bfloat16), wbig_ref[...], (((1,), (0,)), ((), ())),
        preferred_element_type=jnp.float32) + bd2big_ref[...]

    inv_sqrt_hd = 1.0 / (_HD ** 0.5)
    s = jax.lax.dot_general(q.astype(jnp.bfloat16), kcbig, cdims,
                            preferred_element_type=jnp.float32)
    s = s * inv_sqrt_hd * bias_all

    rown = jax.lax.broadcasted_iota(jnp.int32, (_N, 1), 0)
    p = jnp.full((_N, 1), _N + 1, jnp.int32)
    for j in range(_NC):
        p = jnp.where(rown == idx_sref[b, j], j, p)
    j32 = jax.lax.broadcasted_iota(jnp.int32, (_N, _HJ), 1) % _NCP
    k_scal = k_sref[0, 0]
    valid = (j32 != p) & ((j32 - (p < j32).astype(jnp.int32)) < k_scal)
    s = jnp.where(valid, s, -1e30)

    expand = (jax.lax.broadcasted_iota(jnp.int32, (_NH, _HJ), 1) // _NCP ==
              jax.lax.broadcasted_iota(jnp.int32, (_NH, _HJ), 0)
              ).astype(jnp.float32)
    m4 = jnp.concatenate(
        [jnp.max(s[:, h * _NCP:(h + 1) * _NCP], axis=1, keepdims=True)
         for h in range(_NH)], axis=1)
    m32 = jax.lax.dot_general(m4, expand, (((1,), (0,)), ((), ())),
                              preferred_element_type=jnp.float32)
    e = jnp.exp(s - m32)
    den4 = jax.lax.dot_general(e, expand, (((1,), (1,)), ((), ())),
                               preferred_element_type=jnp.float32)
    r32 = jax.lax.dot_general(1.0 / den4, expand, (((1,), (0,)), ((), ())),
                              preferred_element_type=jnp.float32)
    a = e * r32

    attn = jax.lax.dot_general(a.astype(jnp.bfloat16), vcbig,
                               (((1,), (0,)), ((), ())),
                               preferred_element_type=jnp.float32)

    xo = x + attn
    mu = jnp.mean(xo, axis=1, keepdims=True)
    var = jnp.mean((xo - mu) * (xo - mu), axis=1, keepdims=True)
    y = (xo - mu) * jax.lax.rsqrt(var + 1e-5)
    out_ref[0] = y * lng_ref[...] + lnb_ref[...]


@functools.partial(jax.jit, static_argnames=())
def kernel(agent_repr_1, ego_distance, ego_mask, ego_speed,
           Wq, bq, Wk, bk, Wv, bv, Weq, beq, Wek, bek, Wev, bev,
           Wd1, bd1, Wd2, bd2, ln_g, ln_b):
    b, n, d = agent_repr_1.shape
    hd4 = Wd1.shape[0]

    w1a_t = jnp.tile(Wd1[:, 0], _NCP).reshape(1, _NCP * hd4)
    w1b_t = jnp.tile(Wd1[:, 1], _NCP).reshape(1, _NCP * hd4)
    bd1_t = jnp.tile(bd1, _NCP).reshape(1, _NCP * hd4)
    wbig = jnp.einsum('ch,jJ->jchJ', Wd2.T,
                      jnp.eye(_NCP, dtype=jnp.float32)
                      ).reshape(_NCP * hd4, _HJ).astype(jnp.bfloat16)
    bd2big = jnp.repeat(bd2, _NCP).reshape(1, _HJ)
    wqe = jnp.concatenate([Wq, Weq], axis=0).astype(jnp.bfloat16)
    wkv = jnp.concatenate([Wk, Wv], axis=0).astype(jnp.bfloat16)

    top_idx, kdvb, k_arr = pl.pallas_call(
        _select_body,
        out_shape=(
            jax.ShapeDtypeStruct((b, _NCP), jnp.int32),
            jax.ShapeDtypeStruct((b, _NCP * hd4), jnp.float32),
            jax.ShapeDtypeStruct((1, 1), jnp.int32),
        ),
    )(ego_distance, ego_speed.reshape(1, b), w1b_t, bd1_t)

    distT = ego_distance.T
    maskT = ego_mask.astype(jnp.float32).T

    full = lambda shape: pl.BlockSpec(shape, lambda i, *_: (0,) * len(shape))
    grid_spec = pltpu.PrefetchScalarGridSpec(
        num_scalar_prefetch=2,
        grid=(b,),
        in_specs=[
            pl.BlockSpec((1, n, d), lambda i, *_: (i, 0, 0)),
            full((n, b)),
            full((n, b)),
            full((b, _H1)),
            full((2 * d, d)), full((1, d)), full((1, d)),
            full((2 * d, d)), full((1, d)), full((1, d)),
            full((1, _H1)),
            full((_H1, _HJ)),
            full((1, _HJ)),
            full((1, d)), full((1, d)),
        ],
        out_specs=pl.BlockSpec((1, n, d), lambda i, *_: (i, 0, 0)),
        scratch_shapes=[pltpu.VMEM((_NCP, d), jnp.float32)],
    )

    out = pl.pallas_call(
        _attn_body,
        grid_spec=grid_spec,
        out_shape=jax.ShapeDtypeStruct((b, n, d), jnp.float32),
    )(top_idx, k_arr,
      agent_repr_1, distT, maskT, kdvb,
      wqe, bq.reshape(1, d), beq.reshape(1, d),
      wkv, bk.reshape(1, d), bv.reshape(1, d),
      w1a_t, wbig, bd2big,
      ln_g.reshape(1, d), ln_b.reshape(1, d))
    return out


# speedup vs baseline: 1.0327x; 1.0327x over previous
import functools

import jax
import jax.numpy as jnp
from jax.experimental import pallas as pl
from jax.experimental.pallas import tpu as pltpu

_B, _N, _D = 64, 256, 256
_NH = 4
_HD = _D // _NH
_THR = 20.0
_KDEF = 4
_KMAX = 6
_NC = 7
_NCP = 8
_HJ = _NH * _NCP
_H1 = _NCP * (_D // 4)


def _select_body(dist_ref, speed_ref, w1b_ref, bd1t_ref,
                 wq_ref, weq_ref, wk_ref, wv_ref, wbigf_ref,
                 idx_ref, kdvb_ref, k_ref,
                 wqe_ref, wkv_ref, wbig_ref):
    wqe_ref[0:_D, :] = wq_ref[...].astype(jnp.bfloat16)
    wqe_ref[_D:2 * _D, :] = weq_ref[...].astype(jnp.bfloat16)
    wkv_ref[0:_D, :] = wk_ref[...].astype(jnp.bfloat16)
    wkv_ref[_D:2 * _D, :] = wv_ref[...].astype(jnp.bfloat16)
    wbig_ref[...] = wbigf_ref[...].astype(jnp.bfloat16)
    d0 = dist_ref[...]
    close = jnp.sum((d0 < _THR).astype(jnp.float32), axis=1, keepdims=True)
    avg_density = jnp.mean(close) / d0.shape[1]
    avg_speed = jnp.mean(speed_ref[...])
    k = _KDEF + (avg_speed > 15.0).astype(jnp.int32)
    k = jnp.minimum(k, _KMAX)
    k = jnp.minimum(k + (avg_density > 0.5).astype(jnp.int32), _KMAX)
    k = jnp.minimum(k, d0.shape[1] - 1)
    k_ref[...] = jnp.full((1, 1), k, jnp.int32)

    hd4 = _D // 4
    iota_n = jax.lax.broadcasted_iota(jnp.int32, d0.shape, 1)
    work = d0
    for j in range(_NC):
        mval = jnp.min(work, axis=1, keepdims=True)
        cand = jnp.where(work == mval, iota_n, d0.shape[1])
        midx = jnp.min(cand, axis=1, keepdims=True)
        idx_ref[:, j:j + 1] = midx
        sl = slice(j * hd4, (j + 1) * hd4)
        kdvb_ref[:, sl] = mval * w1b_ref[:, sl] + bd1t_ref[:, sl]
        work = jnp.where(iota_n == midx, jnp.inf, work)
    idx_ref[:, _NC:_NCP] = jnp.zeros((_B, _NCP - _NC), jnp.int32)
    sl = slice(_NC * hd4, _NCP * hd4)
    kdvb_ref[:, sl] = jnp.broadcast_to(bd1t_ref[:, sl], (_B, hd4))


def _attn_body(idx_sref, k_sref,
               x_ref, distT_ref, maskT_ref, kdvb_ref,
               wqe_ref, bq_ref, beq_ref,
               wkv_ref, bk_ref, bv_ref,
               w1a_ref, wbig_ref, bd2big_ref,
               lng_ref, lnb_ref, out_ref, cand_ref):
    b = pl.program_id(0)
    x = x_ref[0]
    cdims = (((1,), (1,)), ((), ()))

    qboth = jax.lax.dot_general(x.astype(jnp.bfloat16), wqe_ref[...], cdims,
                                preferred_element_type=jnp.float32)
    qx = qboth[:, :_D] + bq_ref[...]
    qe = qboth[:, _D:] + beq_ref[...]
    lane = jax.lax.broadcasted_iota(jnp.int32, (_N, _B), 1)
    mcol = jnp.sum(jnp.where(lane == b, maskT_ref[...], 0.0),
                   axis=1, keepdims=True)
    q = qx + mcol * (qe - qx)

    for j in range(_NC):
        cand_ref[j:j + 1, :] = x_ref[0, pl.ds(idx_sref[b, j], 1), :]
    cand_ref[_NC:_NCP, :] = jnp.zeros((_NCP - _NC, _D), jnp.float32)
    cand = cand_ref[...]

    kvboth = jax.lax.dot_general(cand.astype(jnp.bfloat16), wkv_ref[...],
                                 cdims, preferred_element_type=jnp.float32)
    kc = kvboth[:, :_D] + bk_ref[...]
    vc = kvboth[:, _D:] + bv_ref[...]

    hol = jax.lax.broadcasted_iota(jnp.int32, (_NCP, _D), 1) // _HD
    kcbig = jnp.concatenate(
        [jnp.where(hol == h, kc, 0.0) for h in range(_NH)],
        axis=0).astype(jnp.bfloat16)
    vcbig = jnp.concatenate(
        [jnp.where(hol == h, vc, 0.0) for h in range(_NH)],
        axis=0).astype(jnp.bfloat16)

    qd = jnp.sum(jnp.where(lane == b, distT_ref[...], 0.0),
                 axis=1, keepdims=True)
    kdvb = kdvb_ref[pl.ds(b, 1), :]
    h_all = jnp.maximum(qd * w1a_ref[...] + kdvb, 0.0)
    bias_all = jax.lax.dot_general(
        h_all.astype(jnp.bfloat16), wbig_ref[...], (((1,), (0,)), ((), ())),
        preferred_element_type=jnp.float32) + bd2big_ref[...]

    inv_sqrt_hd = 1.0 / (_HD ** 0.5)
    s = jax.lax.dot_general(q.astype(jnp.bfloat16), kcbig, cdims,
                            preferred_element_type=jnp.float32)
    s = s * inv_sqrt_hd * bias_all

    rown = jax.lax.broadcasted_iota(jnp.int32, (_N, 1), 0)
    p = jnp.full((_N, 1), _N + 1, jnp.int32)
    for j in range(_NC):
        p = jnp.where(rown == idx_sref[b, j], j, p)
    j32 = jax.lax.broadcasted_iota(jnp.int32, (_N, _HJ), 1) % _NCP
    k_scal = k_sref[0, 0]
    valid = (j32 != p) & ((j32 - (p < j32).astype(jnp.int32)) < k_scal)
    s = jnp.where(valid, s, -1e30)

    expand = (jax.lax.broadcasted_iota(jnp.int32, (_NH, _HJ), 1) // _NCP ==
              jax.lax.broadcasted_iota(jnp.int32, (_NH, _HJ), 0)
              ).astype(jnp.float32)
    m4 = jnp.concatenate(
        [jnp.max(s[:, h * _NCP:(h + 1) * _NCP], axis=1, keepdims=True)
         for h in range(_NH)], axis=1)
    m32 = jax.lax.dot_general(m4, expand, (((1,), (0,)), ((), ())),
                              preferred_element_type=jnp.float32)
    e = jnp.exp(s - m32)
    den4 = jax.lax.dot_general(e, expand, (((1,), (1,)), ((), ())),
                               preferred_element_type=jnp.float32)
    r32 = jax.lax.dot_general(1.0 / den4, expand, (((1,), (0,)), ((), ())),
                              preferred_element_type=jnp.float32)
    a = e * r32

    attn = jax.lax.dot_general(a.astype(jnp.bfloat16), vcbig,
                               (((1,), (0,)), ((), ())),
                               preferred_element_type=jnp.float32)

    xo = x + attn
    mu = jnp.mean(xo, axis=1, keepdims=True)
    var = jnp.mean((xo - mu) * (xo - mu), axis=1, keepdims=True)
    y = (xo - mu) * jax.lax.rsqrt(var + 1e-5)
    out_ref[0] = y * lng_ref[...] + lnb_ref[...]


@functools.partial(jax.jit, static_argnames=())
def kernel(agent_repr_1, ego_distance, ego_mask, ego_speed,
           Wq, bq, Wk, bk, Wv, bv, Weq, beq, Wek, bek, Wev, bev,
           Wd1, bd1, Wd2, bd2, ln_g, ln_b):
    b, n, d = agent_repr_1.shape
    hd4 = Wd1.shape[0]

    w1a_t = jnp.tile(Wd1[:, 0], _NCP).reshape(1, _NCP * hd4)
    w1b_t = jnp.tile(Wd1[:, 1], _NCP).reshape(1, _NCP * hd4)
    bd1_t = jnp.tile(bd1, _NCP).reshape(1, _NCP * hd4)
    wbigf = jnp.einsum('ch,jJ->jchJ', Wd2.T,
                       jnp.eye(_NCP, dtype=jnp.float32)
                       ).reshape(_NCP * hd4, _HJ)
    bd2big = jnp.repeat(bd2, _NCP).reshape(1, _HJ)

    top_idx, kdvb, k_arr, wqe, wkv, wbig = pl.pallas_call(
        _select_body,
        out_shape=(
            jax.ShapeDtypeStruct((b, _NCP), jnp.int32),
            jax.ShapeDtypeStruct((b, _NCP * hd4), jnp.float32),
            jax.ShapeDtypeStruct((1, 1), jnp.int32),
            jax.ShapeDtypeStruct((2 * d, d), jnp.bfloat16),
            jax.ShapeDtypeStruct((2 * d, d), jnp.bfloat16),
            jax.ShapeDtypeStruct((_NCP * hd4, _HJ), jnp.bfloat16),
        ),
    )(ego_distance, ego_speed.reshape(1, b), w1b_t, bd1_t,
      Wq, Weq, Wk, Wv, wbigf)

    distT = ego_distance.T
    maskT = ego_mask.astype(jnp.float32).T

    full = lambda shape: pl.BlockSpec(shape, lambda i, *_: (0,) * len(shape))
    grid_spec = pltpu.PrefetchScalarGridSpec(
        num_scalar_prefetch=2,
        grid=(b,),
        in_specs=[
            pl.BlockSpec((1, n, d), lambda i, *_: (i, 0, 0)),
            full((n, b)),
            full((n, b)),
            full((b, _H1)),
            full((2 * d, d)), full((1, d)), full((1, d)),
            full((2 * d, d)), full((1, d)), full((1, d)),
            full((1, _H1)),
            full((_H1, _HJ)),
            full((1, _HJ)),
            full((1, d)), full((1, d)),
        ],
        out_specs=pl.BlockSpec((1, n, d), lambda i, *_: (i, 0, 0)),
        scratch_shapes=[pltpu.VMEM((_NCP, d), jnp.float32)],
    )

    out = pl.pallas_call(
        _attn_body,
        grid_spec=grid_spec,
        out_shape=jax.ShapeDtypeStruct((b, n, d), jnp.float32),
    )(top_idx, k_arr,
      agent_repr_1, distT, maskT, kdvb,
      wqe, bq.reshape(1, d), beq.reshape(1, d),
      wkv, bk.reshape(1, d), bv.reshape(1, d),
      w1a_t, wbig, bd2big,
      ln_g.reshape(1, d), ln_b.reshape(1, d))
    return out


# speedup vs baseline: 1.6748x; 1.6218x over previous
import functools

import jax
import jax.numpy as jnp
from jax.experimental import pallas as pl
from jax.experimental.pallas import tpu as pltpu

_B, _N, _D = 64, 256, 256
_NH = 4
_HD = _D // _NH
_THR = 20.0
_KDEF = 4
_KMAX = 6
_NC = 7
_NCP = 8
_HJ = _NH * _NCP
_H1 = _NCP * (_D // 4)
_BB = 4


def _select_body(dist_ref, speed_ref, w1b_ref, bd1t_ref,
                 wq_ref, weq_ref, wk_ref, wv_ref, wbigf_ref,
                 idx_ref, kdvb_ref, k_ref,
                 wqe_ref, wkv_ref, wbig_ref):
    wqe_ref[0:_D, :] = wq_ref[...].astype(jnp.bfloat16)
    wqe_ref[_D:2 * _D, :] = weq_ref[...].astype(jnp.bfloat16)
    wkv_ref[0:_D, :] = wk_ref[...].astype(jnp.bfloat16)
    wkv_ref[_D:2 * _D, :] = wv_ref[...].astype(jnp.bfloat16)
    wbig_ref[...] = wbigf_ref[...].astype(jnp.bfloat16)
    d0 = dist_ref[...]
    close = jnp.sum((d0 < _THR).astype(jnp.float32), axis=1, keepdims=True)
    avg_density = jnp.mean(close) / d0.shape[1]
    avg_speed = jnp.mean(speed_ref[...])
    k = _KDEF + (avg_speed > 15.0).astype(jnp.int32)
    k = jnp.minimum(k, _KMAX)
    k = jnp.minimum(k + (avg_density > 0.5).astype(jnp.int32), _KMAX)
    k = jnp.minimum(k, d0.shape[1] - 1)
    k_ref[...] = jnp.full((1, 1), k, jnp.int32)

    hd4 = _D // 4
    iota_n = jax.lax.broadcasted_iota(jnp.int32, d0.shape, 1)
    work = d0
    for j in range(_NC):
        mval = jnp.min(work, axis=1, keepdims=True)
        cand = jnp.where(work == mval, iota_n, d0.shape[1])
        midx = jnp.min(cand, axis=1, keepdims=True)
        idx_ref[:, j:j + 1] = midx
        sl = slice(j * hd4, (j + 1) * hd4)
        kdvb_ref[:, sl] = mval * w1b_ref[:, sl] + bd1t_ref[:, sl]
        work = jnp.where(iota_n == midx, jnp.inf, work)
    idx_ref[:, _NC:_NCP] = jnp.zeros((_B, _NCP - _NC), jnp.int32)
    sl = slice(_NC * hd4, _NCP * hd4)
    kdvb_ref[:, sl] = jnp.broadcast_to(bd1t_ref[:, sl], (_B, hd4))


def _attn_body(idx_sref, k_sref,
               x_ref, distT_ref, maskT_ref, kdvb_ref,
               wqe_ref, bq_ref, beq_ref,
               wkv_ref, bk_ref, bv_ref,
               w1a_ref, wbig_ref, bd2big_ref,
               lng_ref, lnb_ref, out_ref, cand_ref):
    i = pl.program_id(0)
    nb = _BB * _N
    x = x_ref[...].reshape(nb, _D)
    cdims = (((1,), (1,)), ((), ()))

    qboth = jax.lax.dot_general(x.astype(jnp.bfloat16), wqe_ref[...], cdims,
                                preferred_element_type=jnp.float32)
    qx = qboth[:, :_D] + bq_ref[...]
    qe = qboth[:, _D:] + beq_ref[...]

    lane = jax.lax.broadcasted_iota(jnp.int32, (_N, _B), 1)
    maskT = maskT_ref[...]
    distT = distT_ref[...]
    mcol = jnp.concatenate(
        [jnp.sum(jnp.where(lane == i * _BB + t, maskT, 0.0),
                 axis=1, keepdims=True) for t in range(_BB)], axis=0)
    qd = jnp.concatenate(
        [jnp.sum(jnp.where(lane == i * _BB + t, distT, 0.0),
                 axis=1, keepdims=True) for t in range(_BB)], axis=0)
    q = qx + mcol * (qe - qx)

    for t in range(_BB):
        for j in range(_NC):
            cand_ref[t * _NCP + j:t * _NCP + j + 1, :] = (
                x_ref[t, pl.ds(idx_sref[i * _BB + t, j], 1), :])
        cand_ref[t * _NCP + _NC:(t + 1) * _NCP, :] = (
            jnp.zeros((_NCP - _NC, _D), jnp.float32))
    cand = cand_ref[...]

    kvboth = jax.lax.dot_general(cand.astype(jnp.bfloat16), wkv_ref[...],
                                 cdims, preferred_element_type=jnp.float32)
    kc = kvboth[:, :_D] + bk_ref[...]
    vc = kvboth[:, _D:] + bv_ref[...]

    hol = jax.lax.broadcasted_iota(jnp.int32, (_NCP, _D), 1) // _HD
    inv_sqrt_hd = 1.0 / (_HD ** 0.5)
    s_parts = []
    vcbigs = []
    for t in range(_BB):
        sl = slice(t * _NCP, (t + 1) * _NCP)
        kcbig = jnp.concatenate(
            [jnp.where(hol == h, kc[sl, :], 0.0) for h in range(_NH)],
            axis=0).astype(jnp.bfloat16)
        vcbigs.append(jnp.concatenate(
            [jnp.where(hol == h, vc[sl, :], 0.0) for h in range(_NH)],
            axis=0).astype(jnp.bfloat16))
        qt = q[t * _N:(t + 1) * _N, :]
        s_parts.append(jax.lax.dot_general(
            qt.astype(jnp.bfloat16), kcbig, cdims,
            preferred_element_type=jnp.float32))
    s = jnp.concatenate(s_parts, axis=0)

    kdvb = kdvb_ref[0]
    kdexp = jnp.broadcast_to(kdvb.reshape(_BB, 1, _H1),
                             (_BB, _N, _H1)).reshape(nb, _H1)
    h_all = jnp.maximum(qd * w1a_ref[...] + kdexp, 0.0)
    bias_all = jax.lax.dot_general(
        h_all.astype(jnp.bfloat16), wbig_ref[...], (((1,), (0,)), ((), ())),
        preferred_element_type=jnp.float32) + bd2big_ref[...]
    s = s * inv_sqrt_hd * bias_all

    rown = jax.lax.broadcasted_iota(jnp.int32, (_N, 1), 0)
    p_parts = []
    for t in range(_BB):
        p = jnp.full((_N, 1), _N + 1, jnp.int32)
        for j in range(_NC):
            p = jnp.where(rown == idx_sref[i * _BB + t, j], j, p)
        p_parts.append(p)
    p = jnp.concatenate(p_parts, axis=0)
    j32 = jax.lax.broadcasted_iota(jnp.int32, (nb, _HJ), 1) % _NCP
    k_scal = k_sref[0, 0]
    valid = (j32 != p) & ((j32 - (p < j32).astype(jnp.int32)) < k_scal)
    s = jnp.where(valid, s, -1e30)

    expand = (jax.lax.broadcasted_iota(jnp.int32, (_NH, _HJ), 1) // _NCP ==
              jax.lax.broadcasted_iota(jnp.int32, (_NH, _HJ), 0)
              ).astype(jnp.float32)
    m4 = jnp.concatenate(
        [jnp.max(s[:, h * _NCP:(h + 1) * _NCP], axis=1, keepdims=True)
         for h in range(_NH)], axis=1)
    m32 = jax.lax.dot_general(m4, expand, (((1,), (0,)), ((), ())),
                              preferred_element_type=jnp.float32)
    e = jnp.exp(s - m32)
    den4 = jax.lax.dot_general(e, expand, (((1,), (1,)), ((), ())),
                               preferred_element_type=jnp.float32)
    r32 = jax.lax.dot_general(1.0 / den4, expand, (((1,), (0,)), ((), ())),
                              preferred_element_type=jnp.float32)
    a = e * r32

    attn = jnp.concatenate(
        [jax.lax.dot_general(
            a[t * _N:(t + 1) * _N, :].astype(jnp.bfloat16), vcbigs[t],
            (((1,), (0,)), ((), ())), preferred_element_type=jnp.float32)
         for t in range(_BB)], axis=0)

    xo = x + attn
    mu = jnp.mean(xo, axis=1, keepdims=True)
    var = jnp.mean((xo - mu) * (xo - mu), axis=1, keepdims=True)
    y = (xo - mu) * jax.lax.rsqrt(var + 1e-5)
    out_ref[...] = (y * lng_ref[...] + lnb_ref[...]).reshape(_BB, _N, _D)


@functools.partial(jax.jit, static_argnames=())
def kernel(agent_repr_1, ego_distance, ego_mask, ego_speed,
           Wq, bq, Wk, bk, Wv, bv, Weq, beq, Wek, bek, Wev, bev,
           Wd1, bd1, Wd2, bd2, ln_g, ln_b):
    b, n, d = agent_repr_1.shape
    hd4 = Wd1.shape[0]

    w1a_t = jnp.tile(Wd1[:, 0], _NCP).reshape(1, _NCP * hd4)
    w1b_t = jnp.tile(Wd1[:, 1], _NCP).reshape(1, _NCP * hd4)
    bd1_t = jnp.tile(bd1, _NCP).reshape(1, _NCP * hd4)
    wbigf = jnp.einsum('ch,jJ->jchJ', Wd2.T,
                       jnp.eye(_NCP, dtype=jnp.float32)
                       ).reshape(_NCP * hd4, _HJ)
    bd2big = jnp.repeat(bd2, _NCP).reshape(1, _HJ)

    top_idx, kdvb, k_arr, wqe, wkv, wbig = pl.pallas_call(
        _select_body,
        out_shape=(
            jax.ShapeDtypeStruct((b, _NCP), jnp.int32),
            jax.ShapeDtypeStruct((b, _NCP * hd4), jnp.float32),
            jax.ShapeDtypeStruct((1, 1), jnp.int32),
            jax.ShapeDtypeStruct((2 * d, d), jnp.bfloat16),
            jax.ShapeDtypeStruct((2 * d, d), jnp.bfloat16),
            jax.ShapeDtypeStruct((_NCP * hd4, _HJ), jnp.bfloat16),
        ),
    )(ego_distance, ego_speed.reshape(1, b), w1b_t, bd1_t,
      Wq, Weq, Wk, Wv, wbigf)

    distT = ego_distance.T
    maskT = ego_mask.astype(jnp.float32).T

    full = lambda shape: pl.BlockSpec(shape, lambda i, *_: (0,) * len(shape))
    grid_spec = pltpu.PrefetchScalarGridSpec(
        num_scalar_prefetch=2,
        grid=(b // _BB,),
        in_specs=[
            pl.BlockSpec((_BB, n, d), lambda i, *_: (i, 0, 0)),
            full((n, b)),
            full((n, b)),
            pl.BlockSpec((1, _BB, _H1), lambda i, *_: (i, 0, 0)),
            full((2 * d, d)), full((1, d)), full((1, d)),
            full((2 * d, d)), full((1, d)), full((1, d)),
            full((1, _H1)),
            full((_H1, _HJ)),
            full((1, _HJ)),
            full((1, d)), full((1, d)),
        ],
        out_specs=pl.BlockSpec((_BB, n, d), lambda i, *_: (i, 0, 0)),
        scratch_shapes=[pltpu.VMEM((_BB * _NCP, d), jnp.float32)],
    )

    out = pl.pallas_call(
        _attn_body,
        grid_spec=grid_spec,
        out_shape=jax.ShapeDtypeStruct((b, n, d), jnp.float32),
    )(top_idx, k_arr,
      agent_repr_1, distT, maskT, kdvb.reshape(b // _BB, _BB, _H1),
      wqe, bq.reshape(1, d), beq.reshape(1, d),
      wkv, bk.reshape(1, d), bv.reshape(1, d),
      w1a_t, wbig, bd2big,
      ln_g.reshape(1, d), ln_b.reshape(1, d))
    return out


# speedup vs baseline: 1.9127x; 1.1420x over previous
import functools

import jax
import jax.numpy as jnp
from jax.experimental import pallas as pl
from jax.experimental.pallas import tpu as pltpu

_B, _N, _D = 64, 256, 256
_NH = 4
_HD = _D // _NH
_THR = 20.0
_KDEF = 4
_KMAX = 6
_NC = 7
_NCP = 8
_HJ = _NH * _NCP
_H1 = _NCP * (_D // 4)
_BB = 8


def _select_body(dist_ref, speed_ref, w1b_ref, bd1t_ref,
                 wq_ref, weq_ref, wk_ref, wv_ref, wbigf_ref,
                 idx_ref, kdvb_ref, k_ref,
                 wqe_ref, wkv_ref, wbig_ref):
    wqe_ref[0:_D, :] = wq_ref[...].astype(jnp.bfloat16)
    wqe_ref[_D:2 * _D, :] = weq_ref[...].astype(jnp.bfloat16)
    wkv_ref[0:_D, :] = wk_ref[...].astype(jnp.bfloat16)
    wkv_ref[_D:2 * _D, :] = wv_ref[...].astype(jnp.bfloat16)
    wbig_ref[...] = wbigf_ref[...].astype(jnp.bfloat16)
    d0 = dist_ref[...]
    close = jnp.sum((d0 < _THR).astype(jnp.float32), axis=1, keepdims=True)
    avg_density = jnp.mean(close) / d0.shape[1]
    avg_speed = jnp.mean(speed_ref[...])
    k = _KDEF + (avg_speed > 15.0).astype(jnp.int32)
    k = jnp.minimum(k, _KMAX)
    k = jnp.minimum(k + (avg_density > 0.5).astype(jnp.int32), _KMAX)
    k = jnp.minimum(k, d0.shape[1] - 1)
    k_ref[...] = jnp.full((1, 1), k, jnp.int32)

    hd4 = _D // 4
    iota_n = jax.lax.broadcasted_iota(jnp.int32, d0.shape, 1)
    work = d0
    for j in range(_NC):
        mval = jnp.min(work, axis=1, keepdims=True)
        cand = jnp.where(work == mval, iota_n, d0.shape[1])
        midx = jnp.min(cand, axis=1, keepdims=True)
        idx_ref[:, j:j + 1] = midx
        sl = slice(j * hd4, (j + 1) * hd4)
        kdvb_ref[:, sl] = mval * w1b_ref[:, sl] + bd1t_ref[:, sl]
        work = jnp.where(iota_n == midx, jnp.inf, work)
    idx_ref[:, _NC:_NCP] = jnp.zeros((_B, _NCP - _NC), jnp.int32)
    sl = slice(_NC * hd4, _NCP * hd4)
    kdvb_ref[:, sl] = jnp.broadcast_to(bd1t_ref[:, sl], (_B, hd4))


def _attn_body(idx_sref, k_sref,
               x_ref, distT_ref, maskT_ref, kdvb_ref,
               wqe_ref, bq_ref, beq_ref,
               wkv_ref, bk_ref, bv_ref,
               w1a_ref, wbig_ref, bd2big_ref,
               lng_ref, lnb_ref, out_ref, cand_ref):
    i = pl.program_id(0)
    nb = _BB * _N
    x = x_ref[...].reshape(nb, _D)
    cdims = (((1,), (1,)), ((), ()))

    qboth = jax.lax.dot_general(x.astype(jnp.bfloat16), wqe_ref[...], cdims,
                                preferred_element_type=jnp.float32)
    qx = qboth[:, :_D] + bq_ref[...]
    qe = qboth[:, _D:] + beq_ref[...]

    lane = jax.lax.broadcasted_iota(jnp.int32, (_N, _B), 1)
    maskT = maskT_ref[...]
    distT = distT_ref[...]
    mcol = jnp.concatenate(
        [jnp.sum(jnp.where(lane == i * _BB + t, maskT, 0.0),
                 axis=1, keepdims=True) for t in range(_BB)], axis=0)
    qd = jnp.concatenate(
        [jnp.sum(jnp.where(lane == i * _BB + t, distT, 0.0),
                 axis=1, keepdims=True) for t in range(_BB)], axis=0)
    q = qx + mcol * (qe - qx)

    for t in range(_BB):
        for j in range(_NC):
            cand_ref[t * _NCP + j:t * _NCP + j + 1, :] = (
                x_ref[t, pl.ds(idx_sref[i * _BB + t, j], 1), :])
        cand_ref[t * _NCP + _NC:(t + 1) * _NCP, :] = (
            jnp.zeros((_NCP - _NC, _D), jnp.float32))
    cand = cand_ref[...]

    kvboth = jax.lax.dot_general(cand.astype(jnp.bfloat16), wkv_ref[...],
                                 cdims, preferred_element_type=jnp.float32)
    kc = kvboth[:, :_D] + bk_ref[...]
    vc = kvboth[:, _D:] + bv_ref[...]

    hol = jax.lax.broadcasted_iota(jnp.int32, (_NCP, _D), 1) // _HD
    inv_sqrt_hd = 1.0 / (_HD ** 0.5)
    s_parts = []
    vcbigs = []
    for t in range(_BB):
        sl = slice(t * _NCP, (t + 1) * _NCP)
        kcbig = jnp.concatenate(
            [jnp.where(hol == h, kc[sl, :], 0.0) for h in range(_NH)],
            axis=0).astype(jnp.bfloat16)
        vcbigs.append(jnp.concatenate(
            [jnp.where(hol == h, vc[sl, :], 0.0) for h in range(_NH)],
            axis=0).astype(jnp.bfloat16))
        qt = q[t * _N:(t + 1) * _N, :]
        s_parts.append(jax.lax.dot_general(
            qt.astype(jnp.bfloat16), kcbig, cdims,
            preferred_element_type=jnp.float32))
    s = jnp.concatenate(s_parts, axis=0)

    kdvb = kdvb_ref[0]
    kdexp = jnp.broadcast_to(kdvb.reshape(_BB, 1, _H1),
                             (_BB, _N, _H1)).reshape(nb, _H1)
    h_all = jnp.maximum(qd * w1a_ref[...] + kdexp, 0.0)
    bias_all = jax.lax.dot_general(
        h_all.astype(jnp.bfloat16), wbig_ref[...], (((1,), (0,)), ((), ())),
        preferred_element_type=jnp.float32) + bd2big_ref[...]
    s = s * inv_sqrt_hd * bias_all

    rown = jax.lax.broadcasted_iota(jnp.int32, (_N, 1), 0)
    p_parts = []
    for t in range(_BB):
        p = jnp.full((_N, 1), _N + 1, jnp.int32)
        for j in range(_NC):
            p = jnp.where(rown == idx_sref[i * _BB + t, j], j, p)
        p_parts.append(p)
    p = jnp.concatenate(p_parts, axis=0)
    j32 = jax.lax.broadcasted_iota(jnp.int32, (nb, _HJ), 1) % _NCP
    k_scal = k_sref[0, 0]
    valid = (j32 != p) & ((j32 - (p < j32).astype(jnp.int32)) < k_scal)
    s = jnp.where(valid, s, -1e30)

    expand = (jax.lax.broadcasted_iota(jnp.int32, (_NH, _HJ), 1) // _NCP ==
              jax.lax.broadcasted_iota(jnp.int32, (_NH, _HJ), 0)
              ).astype(jnp.float32)
    m4 = jnp.concatenate(
        [jnp.max(s[:, h * _NCP:(h + 1) * _NCP], axis=1, keepdims=True)
         for h in range(_NH)], axis=1)
    m32 = jax.lax.dot_general(m4, expand, (((1,), (0,)), ((), ())),
                              preferred_element_type=jnp.float32)
    e = jnp.exp(s - m32)
    den4 = jax.lax.dot_general(e, expand, (((1,), (1,)), ((), ())),
                               preferred_element_type=jnp.float32)
    r32 = jax.lax.dot_general(1.0 / den4, expand, (((1,), (0,)), ((), ())),
                              preferred_element_type=jnp.float32)
    a = e * r32

    attn = jnp.concatenate(
        [jax.lax.dot_general(
            a[t * _N:(t + 1) * _N, :].astype(jnp.bfloat16), vcbigs[t],
            (((1,), (0,)), ((), ())), preferred_element_type=jnp.float32)
         for t in range(_BB)], axis=0)

    xo = x + attn
    mu = jnp.mean(xo, axis=1, keepdims=True)
    var = jnp.mean((xo - mu) * (xo - mu), axis=1, keepdims=True)
    y = (xo - mu) * jax.lax.rsqrt(var + 1e-5)
    out_ref[...] = (y * lng_ref[...] + lnb_ref[...]).reshape(_BB, _N, _D)


@functools.partial(jax.jit, static_argnames=())
def kernel(agent_repr_1, ego_distance, ego_mask, ego_speed,
           Wq, bq, Wk, bk, Wv, bv, Weq, beq, Wek, bek, Wev, bev,
           Wd1, bd1, Wd2, bd2, ln_g, ln_b):
    b, n, d = agent_repr_1.shape
    hd4 = Wd1.shape[0]

    w1a_t = jnp.tile(Wd1[:, 0], _NCP).reshape(1, _NCP * hd4)
    w1b_t = jnp.tile(Wd1[:, 1], _NCP).reshape(1, _NCP * hd4)
    bd1_t = jnp.tile(bd1, _NCP).reshape(1, _NCP * hd4)
    wbigf = jnp.einsum('ch,jJ->jchJ', Wd2.T,
                       jnp.eye(_NCP, dtype=jnp.float32)
                       ).reshape(_NCP * hd4, _HJ)
    bd2big = jnp.repeat(bd2, _NCP).reshape(1, _HJ)

    top_idx, kdvb, k_arr, wqe, wkv, wbig = pl.pallas_call(
        _select_body,
        out_shape=(
            jax.ShapeDtypeStruct((b, _NCP), jnp.int32),
            jax.ShapeDtypeStruct((b, _NCP * hd4), jnp.float32),
            jax.ShapeDtypeStruct((1, 1), jnp.int32),
            jax.ShapeDtypeStruct((2 * d, d), jnp.bfloat16),
            jax.ShapeDtypeStruct((2 * d, d), jnp.bfloat16),
            jax.ShapeDtypeStruct((_NCP * hd4, _HJ), jnp.bfloat16),
        ),
    )(ego_distance, ego_speed.reshape(1, b), w1b_t, bd1_t,
      Wq, Weq, Wk, Wv, wbigf)

    distT = ego_distance.T
    maskT = ego_mask.astype(jnp.float32).T

    full = lambda shape: pl.BlockSpec(shape, lambda i, *_: (0,) * len(shape))
    grid_spec = pltpu.PrefetchScalarGridSpec(
        num_scalar_prefetch=2,
        grid=(b // _BB,),
        in_specs=[
            pl.BlockSpec((_BB, n, d), lambda i, *_: (i, 0, 0)),
            full((n, b)),
            full((n, b)),
            pl.BlockSpec((1, _BB, _H1), lambda i, *_: (i, 0, 0)),
            full((2 * d, d)), full((1, d)), full((1, d)),
            full((2 * d, d)), full((1, d)), full((1, d)),
            full((1, _H1)),
            full((_H1, _HJ)),
            full((1, _HJ)),
            full((1, d)), full((1, d)),
        ],
        out_specs=pl.BlockSpec((_BB, n, d), lambda i, *_: (i, 0, 0)),
        scratch_shapes=[pltpu.VMEM((_BB * _NCP, d), jnp.float32)],
    )

    out = pl.pallas_call(
        _attn_body,
        grid_spec=grid_spec,
        out_shape=jax.ShapeDtypeStruct((b, n, d), jnp.float32),
    )(top_idx, k_arr,
      agent_repr_1, distT, maskT, kdvb.reshape(b // _BB, _BB, _H1),
      wqe, bq.reshape(1, d), beq.reshape(1, d),
      wkv, bk.reshape(1, d), bv.reshape(1, d),
      w1a_t, wbig, bd2big,
      ln_g.reshape(1, d), ln_b.reshape(1, d))
    return out


# speedup vs baseline: 1.9317x; 1.0100x over previous
import functools

import jax
import jax.numpy as jnp
from jax.experimental import pallas as pl
from jax.experimental.pallas import tpu as pltpu

_B, _N, _D = 64, 256, 256
_NH = 4
_HD = _D // _NH
_THR = 20.0
_KDEF = 4
_KMAX = 6
_NC = 7
_NCP = 8
_HJ = _NH * _NCP
_H1 = _NCP * (_D // 4)
_BB = 16


def _select_body(dist_ref, speed_ref, w1b_ref, bd1t_ref,
                 wq_ref, weq_ref, wk_ref, wv_ref, wbigf_ref,
                 idx_ref, kdvb_ref, k_ref,
                 wqe_ref, wkv_ref, wbig_ref):
    wqe_ref[0:_D, :] = wq_ref[...].astype(jnp.bfloat16)
    wqe_ref[_D:2 * _D, :] = weq_ref[...].astype(jnp.bfloat16)
    wkv_ref[0:_D, :] = wk_ref[...].astype(jnp.bfloat16)
    wkv_ref[_D:2 * _D, :] = wv_ref[...].astype(jnp.bfloat16)
    wbig_ref[...] = wbigf_ref[...].astype(jnp.bfloat16)
    d0 = dist_ref[...]
    close = jnp.sum((d0 < _THR).astype(jnp.float32), axis=1, keepdims=True)
    avg_density = jnp.mean(close) / d0.shape[1]
    avg_speed = jnp.mean(speed_ref[...])
    k = _KDEF + (avg_speed > 15.0).astype(jnp.int32)
    k = jnp.minimum(k, _KMAX)
    k = jnp.minimum(k + (avg_density > 0.5).astype(jnp.int32), _KMAX)
    k = jnp.minimum(k, d0.shape[1] - 1)
    k_ref[...] = jnp.full((1, 1), k, jnp.int32)

    hd4 = _D // 4
    iota_n = jax.lax.broadcasted_iota(jnp.int32, d0.shape, 1)
    work = d0
    for j in range(_NC):
        mval = jnp.min(work, axis=1, keepdims=True)
        cand = jnp.where(work == mval, iota_n, d0.shape[1])
        midx = jnp.min(cand, axis=1, keepdims=True)
        idx_ref[:, j:j + 1] = midx
        sl = slice(j * hd4, (j + 1) * hd4)
        kdvb_ref[:, sl] = mval * w1b_ref[:, sl] + bd1t_ref[:, sl]
        work = jnp.where(iota_n == midx, jnp.inf, work)
    idx_ref[:, _NC:_NCP] = jnp.zeros((_B, _NCP - _NC), jnp.int32)
    sl = slice(_NC * hd4, _NCP * hd4)
    kdvb_ref[:, sl] = jnp.broadcast_to(bd1t_ref[:, sl], (_B, hd4))


def _attn_body(idx_sref, k_sref,
               x_ref, distT_ref, maskT_ref, kdvb_ref,
               wqe_ref, bq_ref, beq_ref,
               wkv_ref, bk_ref, bv_ref,
               w1a_ref, wbig_ref, bd2big_ref,
               lng_ref, lnb_ref, out_ref, cand_ref):
    i = pl.program_id(0)
    nb = _BB * _N
    x = x_ref[...].reshape(nb, _D)
    cdims = (((1,), (1,)), ((), ()))

    qboth = jax.lax.dot_general(x.astype(jnp.bfloat16), wqe_ref[...], cdims,
                                preferred_element_type=jnp.float32)
    qx = qboth[:, :_D] + bq_ref[...]
    qe = qboth[:, _D:] + beq_ref[...]

    lane = jax.lax.broadcasted_iota(jnp.int32, (_N, _B), 1)
    maskT = maskT_ref[...]
    distT = distT_ref[...]
    mcol = jnp.concatenate(
        [jnp.sum(jnp.where(lane == i * _BB + t, maskT, 0.0),
                 axis=1, keepdims=True) for t in range(_BB)], axis=0)
    qd = jnp.concatenate(
        [jnp.sum(jnp.where(lane == i * _BB + t, distT, 0.0),
                 axis=1, keepdims=True) for t in range(_BB)], axis=0)
    q = qx + mcol * (qe - qx)

    for t in range(_BB):
        for j in range(_NC):
            cand_ref[t * _NCP + j:t * _NCP + j + 1, :] = (
                x_ref[t, pl.ds(idx_sref[i * _BB + t, j], 1), :])
        cand_ref[t * _NCP + _NC:(t + 1) * _NCP, :] = (
            jnp.zeros((_NCP - _NC, _D), jnp.float32))
    cand = cand_ref[...]

    kvboth = jax.lax.dot_general(cand.astype(jnp.bfloat16), wkv_ref[...],
                                 cdims, preferred_element_type=jnp.float32)
    kc = kvboth[:, :_D] + bk_ref[...]
    vc = kvboth[:, _D:] + bv_ref[...]

    hol = jax.lax.broadcasted_iota(jnp.int32, (_NCP, _D), 1) // _HD
    inv_sqrt_hd = 1.0 / (_HD ** 0.5)
    s_parts = []
    vcbigs = []
    for t in range(_BB):
        sl = slice(t * _NCP, (t + 1) * _NCP)
        kcbig = jnp.concatenate(
            [jnp.where(hol == h, kc[sl, :], 0.0) for h in range(_NH)],
            axis=0).astype(jnp.bfloat16)
        vcbigs.append(jnp.concatenate(
            [jnp.where(hol == h, vc[sl, :], 0.0) for h in range(_NH)],
            axis=0).astype(jnp.bfloat16))
        qt = q[t * _N:(t + 1) * _N, :]
        s_parts.append(jax.lax.dot_general(
            qt.astype(jnp.bfloat16), kcbig, cdims,
            preferred_element_type=jnp.float32))
    s = jnp.concatenate(s_parts, axis=0)

    kdvb = kdvb_ref[0]
    kdexp = jnp.broadcast_to(kdvb.reshape(_BB, 1, _H1),
                             (_BB, _N, _H1)).reshape(nb, _H1)
    h_all = jnp.maximum(qd * w1a_ref[...] + kdexp, 0.0)
    bias_all = jax.lax.dot_general(
        h_all.astype(jnp.bfloat16), wbig_ref[...], (((1,), (0,)), ((), ())),
        preferred_element_type=jnp.float32) + bd2big_ref[...]
    s = s * inv_sqrt_hd * bias_all

    rown = jax.lax.broadcasted_iota(jnp.int32, (_N, 1), 0)
    p_parts = []
    for t in range(_BB):
        p = jnp.full((_N, 1), _N + 1, jnp.int32)
        for j in range(_NC):
            p = jnp.where(rown == idx_sref[i * _BB + t, j], j, p)
        p_parts.append(p)
    p = jnp.concatenate(p_parts, axis=0)
    j32 = jax.lax.broadcasted_iota(jnp.int32, (nb, _HJ), 1) % _NCP
    k_scal = k_sref[0, 0]
    valid = (j32 != p) & ((j32 - (p < j32).astype(jnp.int32)) < k_scal)
    s = jnp.where(valid, s, -1e30)

    expand = (jax.lax.broadcasted_iota(jnp.int32, (_NH, _HJ), 1) // _NCP ==
              jax.lax.broadcasted_iota(jnp.int32, (_NH, _HJ), 0)
              ).astype(jnp.float32)
    m4 = jnp.concatenate(
        [jnp.max(s[:, h * _NCP:(h + 1) * _NCP], axis=1, keepdims=True)
         for h in range(_NH)], axis=1)
    m32 = jax.lax.dot_general(m4, expand, (((1,), (0,)), ((), ())),
                              preferred_element_type=jnp.float32)
    e = jnp.exp(s - m32)
    den4 = jax.lax.dot_general(e, expand, (((1,), (1,)), ((), ())),
                               preferred_element_type=jnp.float32)
    r32 = jax.lax.dot_general(1.0 / den4, expand, (((1,), (0,)), ((), ())),
                              preferred_element_type=jnp.float32)
    a = e * r32

    attn = jnp.concatenate(
        [jax.lax.dot_general(
            a[t * _N:(t + 1) * _N, :].astype(jnp.bfloat16), vcbigs[t],
            (((1,), (0,)), ((), ())), preferred_element_type=jnp.float32)
         for t in range(_BB)], axis=0)

    xo = x + attn
    mu = jnp.mean(xo, axis=1, keepdims=True)
    var = jnp.mean((xo - mu) * (xo - mu), axis=1, keepdims=True)
    y = (xo - mu) * jax.lax.rsqrt(var + 1e-5)
    out_ref[...] = (y * lng_ref[...] + lnb_ref[...]).reshape(_BB, _N, _D)


@functools.partial(jax.jit, static_argnames=())
def kernel(agent_repr_1, ego_distance, ego_mask, ego_speed,
           Wq, bq, Wk, bk, Wv, bv, Weq, beq, Wek, bek, Wev, bev,
           Wd1, bd1, Wd2, bd2, ln_g, ln_b):
    b, n, d = agent_repr_1.shape
    hd4 = Wd1.shape[0]

    w1a_t = jnp.tile(Wd1[:, 0], _NCP).reshape(1, _NCP * hd4)
    w1b_t = jnp.tile(Wd1[:, 1], _NCP).reshape(1, _NCP * hd4)
    bd1_t = jnp.tile(bd1, _NCP).reshape(1, _NCP * hd4)
    wbigf = jnp.einsum('ch,jJ->jchJ', Wd2.T,
                       jnp.eye(_NCP, dtype=jnp.float32)
                       ).reshape(_NCP * hd4, _HJ)
    bd2big = jnp.repeat(bd2, _NCP).reshape(1, _HJ)

    top_idx, kdvb, k_arr, wqe, wkv, wbig = pl.pallas_call(
        _select_body,
        out_shape=(
            jax.ShapeDtypeStruct((b, _NCP), jnp.int32),
            jax.ShapeDtypeStruct((b, _NCP * hd4), jnp.float32),
            jax.ShapeDtypeStruct((1, 1), jnp.int32),
            jax.ShapeDtypeStruct((2 * d, d), jnp.bfloat16),
            jax.ShapeDtypeStruct((2 * d, d), jnp.bfloat16),
            jax.ShapeDtypeStruct((_NCP * hd4, _HJ), jnp.bfloat16),
        ),
    )(ego_distance, ego_speed.reshape(1, b), w1b_t, bd1_t,
      Wq, Weq, Wk, Wv, wbigf)

    distT = ego_distance.T
    maskT = ego_mask.astype(jnp.float32).T

    full = lambda shape: pl.BlockSpec(shape, lambda i, *_: (0,) * len(shape))
    grid_spec = pltpu.PrefetchScalarGridSpec(
        num_scalar_prefetch=2,
        grid=(b // _BB,),
        in_specs=[
            pl.BlockSpec((_BB, n, d), lambda i, *_: (i, 0, 0)),
            full((n, b)),
            full((n, b)),
            pl.BlockSpec((1, _BB, _H1), lambda i, *_: (i, 0, 0)),
            full((2 * d, d)), full((1, d)), full((1, d)),
            full((2 * d, d)), full((1, d)), full((1, d)),
            full((1, _H1)),
            full((_H1, _HJ)),
            full((1, _HJ)),
            full((1, d)), full((1, d)),
        ],
        out_specs=pl.BlockSpec((_BB, n, d), lambda i, *_: (i, 0, 0)),
        scratch_shapes=[pltpu.VMEM((_BB * _NCP, d), jnp.float32)],
    )

    out = pl.pallas_call(
        _attn_body,
        grid_spec=grid_spec,
        out_shape=jax.ShapeDtypeStruct((b, n, d), jnp.float32),
    )(top_idx, k_arr,
      agent_repr_1, distT, maskT, kdvb.reshape(b // _BB, _BB, _H1),
      wqe, bq.reshape(1, d), beq.reshape(1, d),
      wkv, bk.reshape(1, d), bv.reshape(1, d),
      w1a_t, wbig, bd2big,
      ln_g.reshape(1, d), ln_b.reshape(1, d))
    return out
